# Initial kernel scaffold; baseline (speedup 1.0000x reference)
import jax
import jax.numpy as jnp
from jax.experimental import pallas as pl


def kernel(pos, embedding, ib_mlp_w1, ib_mlp_b1, ib_mlp_w2, ib_mlp_b2, ib_conv_lin1_w, ib_conv_lin2_w, ib_conv_lin2_b, ib_lin_w, ib_lin_b, lin1_w, lin1_b, lin2_w, lin2_b, vw1, vb1, vw2, vb2, vw3, vb3, vw4, vb4, vw5, vb5, vw6, vb6):
    raise NotImplementedError("write your pallas kernel here")


# speedup vs baseline: 27.9441x; 27.9441x over previous
import math

import jax
import jax.numpy as jnp
from jax.experimental import pallas as pl
from jax.experimental.pallas import tpu as pltpu

HC = 128
NF = 128
NI = 6
NG = 50
CUTOFF = 10.0
NUM_NODES = 10
OS = 6
HS = 256
_LOG2 = math.log(2.0)
_GAP = CUTOFF / (NG - 1)
_COEFF = -0.5 / (_GAP * _GAP)

_PAIRS = [(a, b) for a in range(NUM_NODES) for b in range(a + 1, NUM_NODES)]
_NPAIR = len(_PAIRS)

_Z_PATTERN = (0, 0, 1, 2, 0, 0, 0, 1, 2, 0)


def _ssp(x):
    return jnp.maximum(x, 0.0) + jnp.log(1.0 + jnp.exp(-jnp.abs(x))) - _LOG2


def _elu(x):
    return jnp.where(x > 0, x, jnp.exp(jnp.minimum(x, 0.0)) - 1.0)


def _body(pos_ref, h0_ref, w1_ref, b1_ref, w2_ref, b2_ref,
          cl1_ref, cl2_ref, cl2b_ref, linw_ref, linb_ref,
          l1w_ref, l1b_ref, l2w_ref, l2b_ref,
          vw1_ref, vb1_ref, vw2_ref, vb2_ref, vw3_ref, vb3_ref,
          vw4_ref, vb4_ref, vw5_ref, vb5_ref, vw6_ref, vb6_ref,
          out_ref, *, mb):
    f32 = jnp.float32
    posb = pos_ref[...]
    offset = jax.lax.broadcasted_iota(jnp.int32, (1, NG), 1).astype(jnp.float32) * _GAP

    ea_list = []
    cm_list = []
    for (a, b) in _PAIRS:
        dx = posb[:, 3 * a:3 * a + 1] - posb[:, 3 * b:3 * b + 1]
        dy = posb[:, 3 * a + 1:3 * a + 2] - posb[:, 3 * b + 1:3 * b + 2]
        dz = posb[:, 3 * a + 2:3 * a + 3] - posb[:, 3 * b + 2:3 * b + 3]
        d = jnp.sqrt(dx * dx + dy * dy + dz * dz)
        ea_list.append(jnp.exp(_COEFF * (d - offset) ** 2))
        c = 0.5 * (jnp.cos(d * (math.pi / CUTOFF)) + 1.0)
        cm_list.append(jnp.where(d < CUTOFF, c, 0.0))
    EA = jnp.concatenate(ea_list, axis=0)

    hs = [jnp.broadcast_to(h0_ref[n:n + 1, :], (mb, HC)) for n in range(NUM_NODES)]

    for k in range(NI):
        t = jnp.dot(EA, w1_ref[k], preferred_element_type=f32) + b1_ref[k]
        W = jnp.dot(_ssp(t), w2_ref[k], preferred_element_type=f32) + b2_ref[k]
        xk = [jnp.dot(h, cl1_ref[k], preferred_element_type=f32) for h in hs]
        acc = [jnp.zeros((mb, NF), f32) for _ in range(NUM_NODES)]
        for p, (a, b) in enumerate(_PAIRS):
            wp = W[p * mb:(p + 1) * mb, :] * cm_list[p]
            acc[a] = acc[a] + wp * xk[b]
            acc[b] = acc[b] + wp * xk[a]
        for n in range(NUM_NODES):
            v = jnp.dot(acc[n], cl2_ref[k], preferred_element_type=f32) + cl2b_ref[k]
            v = jnp.dot(_ssp(v), linw_ref[k], preferred_element_type=f32) + linb_ref[k]
            hs[n] = hs[n] + v

    x = None
    for n in range(NUM_NODES):
        t = _ssp(jnp.dot(hs[n], l1w_ref[...], preferred_element_type=f32) + l1b_ref[...])
        hn = jnp.dot(t, l2w_ref[...], preferred_element_type=f32) + l2b_ref[...]
        contrib = jnp.dot(hn, vw1_ref[n], preferred_element_type=f32)
        x = contrib if x is None else x + contrib
    x = _elu(x + vb1_ref[...])
    x = _elu(jnp.dot(x, vw2_ref[...], preferred_element_type=f32) + vb2_ref[...])
    x = _elu(jnp.dot(x, vw3_ref[...], preferred_element_type=f32) + vb3_ref[...])
    x = _elu(jnp.dot(x, vw4_ref[...], preferred_element_type=f32) + vb4_ref[...])
    x = _elu(jnp.dot(x, vw5_ref[...], preferred_element_type=f32) + vb5_ref[...])
    logits = jnp.dot(x, vw6_ref[...], preferred_element_type=f32) + vb6_ref[...]
    lane = jax.lax.broadcasted_iota(jnp.int32, (mb, 128), 1)
    valid = lane < OS
    logits = jnp.where(valid, logits, -1e30)
    m = jnp.max(logits, axis=1, keepdims=True)
    e = jnp.where(valid, jnp.exp(logits - m), 0.0)
    out_ref[...] = e / jnp.sum(e, axis=1, keepdims=True)


def kernel(pos, embedding, ib_mlp_w1, ib_mlp_b1, ib_mlp_w2, ib_mlp_b2,
           ib_conv_lin1_w, ib_conv_lin2_w, ib_conv_lin2_b, ib_lin_w, ib_lin_b,
           lin1_w, lin1_b, lin2_w, lin2_b,
           vw1, vb1, vw2, vb2, vw3, vb3, vw4, vb4, vw5, vb5, vw6, vb6):
    n = pos.shape[0]
    bsz = n // NUM_NODES
    mb = 200 if bsz % 200 == 0 else (8 if bsz % 8 == 0 else 1)
    grid = (bsz // mb,)

    posr = pos.reshape(bsz, NUM_NODES * 3)
    h0 = embedding[jnp.array(_Z_PATTERN, dtype=jnp.int32)]
    h0 = jnp.concatenate([h0, jnp.zeros((6, HC), h0.dtype)], axis=0)
    vw1r = vw1.reshape(NUM_NODES, HC, HS)
    vw6p = jnp.concatenate([vw6, jnp.zeros((HS, 128 - OS), vw6.dtype)], axis=1)
    vb6p = jnp.concatenate([vb6, jnp.zeros((128 - OS,), vb6.dtype)]).reshape(1, 128)

    operands = [
        posr, h0,
        ib_mlp_w1, ib_mlp_b1.reshape(NI, 1, NF),
        ib_mlp_w2, ib_mlp_b2.reshape(NI, 1, NF),
        ib_conv_lin1_w, ib_conv_lin2_w, ib_conv_lin2_b.reshape(NI, 1, HC),
        ib_lin_w, ib_lin_b.reshape(NI, 1, HC),
        lin1_w, lin1_b.reshape(1, HC // 2), lin2_w, lin2_b.reshape(1, HC),
        vw1r, vb1.reshape(1, HS), vw2, vb2.reshape(1, HS),
        vw3, vb3.reshape(1, HS), vw4, vb4.reshape(1, HS),
        vw5, vb5.reshape(1, HS), vw6p, vb6p,
    ]

    def const_spec(arr):
        nd = arr.ndim
        return pl.BlockSpec(arr.shape, lambda i, _nd=nd: (0,) * _nd)

    in_specs = [pl.BlockSpec((mb, NUM_NODES * 3), lambda i: (i, 0))]
    in_specs += [const_spec(a) for a in operands[1:]]

    import functools
    out = pl.pallas_call(
        functools.partial(_body, mb=mb),
        grid=grid,
        in_specs=in_specs,
        out_specs=pl.BlockSpec((mb, 128), lambda i: (i, 0)),
        out_shape=jax.ShapeDtypeStruct((bsz, 128), jnp.float32),
        compiler_params=pltpu.CompilerParams(
            dimension_semantics=("arbitrary",),
        ),
    )(*operands)
    return out[:, :OS]
